# baseline (speedup 1.0000x reference)
import functools

import jax
import jax.numpy as jnp
import numpy as np
from jax import lax
from jax.experimental import pallas as pl
from jax.experimental.pallas import tpu as pltpu
from jax.experimental.pallas import tpu_sc as plsc

SEQ = 512
DIAG = 1
DMODEL = 64
NPAIR = SEQ * (SEQ - 1) // 2
BATCH = 2
TOTAL = BATCH * NPAIR
NWORK = 32
ROWS_PW = TOTAL // NWORK
CHUNK = 112
NCHUNK = ROWS_PW // CHUNK


def _build_indices() -> np.ndarray:
    iu, ju = np.triu_indices(SEQ, DIAG)
    flat = (iu + SEQ * ju).astype(np.int32)
    both = np.concatenate([flat, flat + SEQ * SEQ])
    return both.reshape(NWORK, NCHUNK, CHUNK)


_IDX = _build_indices()


def _sc_body(table_hbm, idx_hbm, out_hbm, idx_v, rows_v, gsem):
    wid = lax.axis_index("s") * 2 + lax.axis_index("c")
    pltpu.sync_copy(idx_hbm.at[wid], idx_v)

    def chunk(c, carry):
        pltpu.async_copy(table_hbm.at[idx_v.at[c]], rows_v, gsem).wait()
        pltpu.sync_copy(rows_v, out_hbm.at[wid, c])
        return carry

    lax.fori_loop(0, NCHUNK, chunk, 0)


@jax.jit
def _gather(table, idx):
    k = functools.partial(
        pl.kernel,
        mesh=plsc.VectorSubcoreMesh(core_axis_name="c", subcore_axis_name="s"),
        out_type=jax.ShapeDtypeStruct((NWORK, NCHUNK, CHUNK, DMODEL), jnp.float32),
        scratch_types=[
            pltpu.VMEM((NCHUNK, CHUNK), jnp.int32),
            pltpu.VMEM((CHUNK, DMODEL), jnp.float32),
            pltpu.SemaphoreType.DMA,
        ],
        compiler_params=pltpu.CompilerParams(use_tc_tiling_on_sc=False),
    )(_sc_body)
    return k(table, idx)


def kernel(inputs):
    table = inputs.reshape(BATCH * SEQ * SEQ, DMODEL)
    idx = jnp.asarray(_IDX)
    out = _gather(table, idx)
    return out.reshape(BATCH, NPAIR, DMODEL)
